# Initial kernel scaffold; baseline (speedup 1.0000x reference)
import jax
import jax.numpy as jnp
from jax.experimental import pallas as pl


def kernel(node_features, frac_coords, lattices_rep, edge_index, edge2graph, num_atoms, W_e1, b_e1, W_e2, b_e2, W_n1, b_n1, W_n2, b_n2):
    raise NotImplementedError("write your pallas kernel here")


# speedup vs baseline: 2.1725x; 2.1725x over previous
import dataclasses
import functools

import jax
import jax.numpy as jnp
from jax import lax
from jax.experimental import pallas as pl
from jax.experimental.pallas import tpu as pltpu
from jax.experimental.pallas import tpu_sc as plsc

N = 10000
E = 320000
G = 500
H = 128

NC = 2
NS = 16
NW = NC * NS
EPW = E // NW
CH = 80
CNTW = 16

_mesh = plsc.VectorSubcoreMesh(core_axis_name="c", subcore_axis_name="s")

_cp = pltpu.CompilerParams()
if "needs_layout_passes" in pltpu.CompilerParams.__dataclass_fields__:
    _cp = dataclasses.replace(_cp, needs_layout_passes=False)

NR = 80
L = 16


def _silu(x):
    return x * jax.nn.sigmoid(x)


def _prep_body(nf_ref, lat_ref, wa_ref, wb_ref, wip_ref, be1_ref,
               p_ref, q_ref, lb_ref):
    nf = nf_ref[...]
    p_ref[...] = jnp.dot(nf, wa_ref[...], preferred_element_type=jnp.float32)
    q_ref[...] = jnp.dot(nf, wb_ref[...], preferred_element_type=jnp.float32)
    x = lat_ref[...]
    lb = be1_ref[...]
    for i in range(3):
        for j in range(3):
            ip = jnp.sum(x[:, 3 * i:3 * i + 3] * x[:, 3 * j:3 * j + 3],
                         axis=1, keepdims=True)
            lb = lb + ip * wip_ref[3 * i + j:3 * i + j + 1, :]
    lb_ref[...] = lb


def _tc_prep(nf, lat9, wa, wb, wip, be1):
    return pl.pallas_call(
        _prep_body,
        out_shape=(
            jax.ShapeDtypeStruct((N, H), jnp.float32),
            jax.ShapeDtypeStruct((N, H), jnp.float32),
            jax.ShapeDtypeStruct((G, H), jnp.float32),
        ),
    )(nf, lat9, wa, wb, wip, be1)


@functools.partial(
    pl.kernel,
    mesh=_mesh,
    out_type=jax.ShapeDtypeStruct((E, H), jnp.float32),
    scratch_types=[
        pltpu.VMEM((CH,), jnp.int32),
        pltpu.VMEM((CH,), jnp.int32),
        pltpu.VMEM((CH,), jnp.int32),
        pltpu.VMEM((CH, H), jnp.float32),
        pltpu.VMEM((CH, H), jnp.float32),
        pltpu.VMEM((CH, H), jnp.float32),
        pltpu.SemaphoreType.DMA,
        pltpu.SemaphoreType.DMA,
        pltpu.SemaphoreType.DMA,
    ],
)
def _sc_gather(p_hbm, q_hbm, lb_hbm, src_hbm, dst_hbm, e2g_hbm, out_hbm,
               si_v, di_v, gi_v, a_v, b_v, c_v, sem_a, sem_b, sem_c):
    wid = lax.axis_index("s") * NC + lax.axis_index("c")
    base = wid * EPW

    @pl.loop(0, EPW, step=CH)
    def _(off):
        b0 = base + off
        pltpu.sync_copy(src_hbm.at[pl.ds(b0, CH)], si_v)
        pltpu.sync_copy(dst_hbm.at[pl.ds(b0, CH)], di_v)
        pltpu.sync_copy(e2g_hbm.at[pl.ds(b0, CH)], gi_v)
        cp_a = pltpu.async_copy(p_hbm.at[si_v], a_v, sem_a)
        cp_b = pltpu.async_copy(q_hbm.at[di_v], b_v, sem_b)
        cp_c = pltpu.async_copy(lb_hbm.at[gi_v], c_v, sem_c)
        cp_a.wait()
        cp_b.wait()
        cp_c.wait()

        @pl.loop(0, CH)
        def _(r):
            for c in range(H // 16):
                slc = (pl.ds(r, 1), pl.ds(c * 16, 16))
                a_v.at[*slc][...] = (a_v.at[*slc][...] + b_v.at[*slc][...]
                                     + c_v.at[*slc][...])

        pltpu.sync_copy(a_v, out_hbm.at[pl.ds(b0, CH)])


BE = 2000


def _emlp_body(pre_ref, w2_ref, b2_ref, out_ref):
    e1 = _silu(pre_ref[...])
    e2 = jnp.dot(e1.astype(jnp.bfloat16), w2_ref[...].astype(jnp.bfloat16),
                 preferred_element_type=jnp.float32) + b2_ref[...]
    out_ref[...] = _silu(e2)


def _tc_edge_mlp(pre, w2, b2):
    return pl.pallas_call(
        _emlp_body,
        grid=(E // BE,),
        in_specs=[
            pl.BlockSpec((BE, H), lambda i: (i, 0)),
            pl.BlockSpec((H, H), lambda i: (0, 0)),
            pl.BlockSpec((1, H), lambda i: (0, 0)),
        ],
        out_specs=pl.BlockSpec((BE, H), lambda i: (i, 0)),
        out_shape=jax.ShapeDtypeStruct((E, H), jnp.float32),
    )(pre, w2, b2)


@functools.partial(
    pl.kernel,
    mesh=_mesh,
    compiler_params=_cp,
    out_type=(
        jax.ShapeDtypeStruct((NC, N, H), jnp.float32),
        jax.ShapeDtypeStruct((NC, NR, H), jnp.float32),
    ),
    scratch_types=[
        pltpu.VMEM_SHARED((N, H), jnp.float32),
        pltpu.VMEM_SHARED((NR, H), jnp.float32),
        pltpu.VMEM((CH,), jnp.int32),
        pltpu.VMEM((CH,), jnp.int32),
        pltpu.VMEM((CH, H), jnp.float32),
        pltpu.VMEM((CH, H), jnp.float32),
    ],
)
def _sc_scatter(ef_hbm, src_hbm, zs_hbm, sums_hbm, cnt_hbm,
                shared_s, shared_c, idx_v, cidx_v, rows_v, oh_v):
    cid = lax.axis_index("c")
    sid = lax.axis_index("s")

    zrp = 624
    pltpu.sync_copy(zs_hbm.at[pl.ds(sid * zrp, zrp)],
                    shared_s.at[pl.ds(sid * zrp, zrp)])

    @pl.when(sid == 0)
    def _():
        ztail = NS * zrp
        pltpu.sync_copy(zs_hbm.at[pl.ds(ztail, N - ztail)],
                        shared_s.at[pl.ds(ztail, N - ztail)])
        pltpu.sync_copy(zs_hbm.at[pl.ds(0, NR)], shared_c)

    pltpu.sync_copy(zs_hbm.at[pl.ds(0, CH)], oh_v)
    plsc.subcore_barrier()

    base = cid * (E // NC) + sid * EPW
    ones16 = jnp.full((L,), 1.0, jnp.float32)
    zeros16 = jnp.zeros((L,), jnp.float32)

    @pl.loop(0, EPW, step=CH)
    def _(off):
        b0 = base + off
        pltpu.sync_copy(src_hbm.at[pl.ds(b0, CH)], idx_v)
        pltpu.sync_copy(ef_hbm.at[pl.ds(b0, CH)], rows_v)
        pltpu.sync_copy(rows_v, shared_s.at[idx_v], add=True)
        for g in range(CH // L):
            v = idx_v[pl.ds(g * L, L)]
            row = lax.iota(jnp.int32, L) + (g * L)
            lane = lax.bitwise_and(v, 127)
            plsc.store_scatter(oh_v, [row, lane], ones16)
            cidx_v.at[pl.ds(g * L, L)][...] = lax.shift_right_logical(v, 7)
        pltpu.sync_copy(oh_v, shared_c.at[cidx_v], add=True)
        for g in range(CH // L):
            v = idx_v[pl.ds(g * L, L)]
            row = lax.iota(jnp.int32, L) + (g * L)
            lane = lax.bitwise_and(v, 127)
            plsc.store_scatter(oh_v, [row, lane], zeros16)

    plsc.subcore_barrier()
    rp = 624
    pltpu.sync_copy(shared_s.at[pl.ds(sid * rp, rp)],
                    sums_hbm.at[cid, pl.ds(sid * rp, rp)])

    @pl.when(sid == 0)
    def _():
        tail = NS * rp
        pltpu.sync_copy(shared_s.at[pl.ds(tail, N - tail)],
                        sums_hbm.at[cid, pl.ds(tail, N - tail)])
        pltpu.sync_copy(shared_c, cnt_hbm.at[cid])


def _nmlp_body(nf_ref, s_ref, c_ref, w1a_ref, w1b_ref, b1_ref, w2_ref, b2_ref,
               out_ref):
    nf = nf_ref[...]
    s = s_ref[0] + s_ref[1]
    cpacked = c_ref[0] + c_ref[1]
    b = jnp.broadcast_to(cpacked[:, None, :], (NR, H, H)).reshape(NR * H, H)
    lane = lax.broadcasted_iota(jnp.int32, (NR * H, H), 1)
    rowm = lax.broadcasted_iota(jnp.int32, (NR * H, H), 0) % H
    cnt = jnp.sum(jnp.where(lane == rowm, b, 0.0), axis=1, keepdims=True)[:N]
    agg = s / jnp.maximum(cnt, 1.0)
    n1 = _silu(jnp.dot(nf.astype(jnp.bfloat16),
                       w1a_ref[...].astype(jnp.bfloat16),
                       preferred_element_type=jnp.float32)
               + jnp.dot(agg.astype(jnp.bfloat16),
                         w1b_ref[...].astype(jnp.bfloat16),
                         preferred_element_type=jnp.float32)
               + b1_ref[...])
    n2 = _silu(jnp.dot(n1.astype(jnp.bfloat16), w2_ref[...].astype(jnp.bfloat16),
                       preferred_element_type=jnp.float32)
               + b2_ref[...])
    out_ref[...] = nf + n2


def _tc_node_mlp(nf, sums, cnt, w1a, w1b, b1, w2, b2):
    return pl.pallas_call(
        _nmlp_body,
        out_shape=jax.ShapeDtypeStruct((N, H), jnp.float32),
    )(nf, sums, cnt, w1a, w1b, b1, w2, b2)


def kernel(node_features, frac_coords, lattices_rep, edge_index, edge2graph,
           num_atoms, W_e1, b_e1, W_e2, b_e2, W_n1, b_n1, W_n2, b_n2):
    del frac_coords, num_atoms
    lat9 = lattices_rep.reshape(G, 9)
    wa = W_e1[:H]
    wb = W_e1[H:2 * H]
    wip = W_e1[2 * H:]
    be1 = b_e1.reshape(1, H)
    src = edge_index[0]
    dst = edge_index[1]

    p, q, lb = _tc_prep(node_features, lat9, wa, wb, wip, be1)
    pre = _sc_gather(p, q, lb, src, dst, edge2graph)
    ef = _tc_edge_mlp(pre, W_e2, b_e2.reshape(1, H))

    zs = jnp.zeros((N, H), jnp.float32)
    sums, cnt = _sc_scatter(ef, src, zs)

    return _tc_node_mlp(node_features, sums, cnt,
                        W_n1[:H], W_n1[H:], b_n1.reshape(1, H),
                        W_n2, b_n2.reshape(1, H))


# speedup vs baseline: 2.7022x; 1.2438x over previous
import dataclasses
import functools

import jax
import jax.numpy as jnp
from jax import lax
from jax.experimental import pallas as pl
from jax.experimental.pallas import tpu as pltpu
from jax.experimental.pallas import tpu_sc as plsc

N = 10000
E = 320000
G = 500
H = 128

NC = 2
NS = 16
NW = NC * NS
EPW = E // NW
CH = 80
CNTW = 16

_mesh = plsc.VectorSubcoreMesh(core_axis_name="c", subcore_axis_name="s")

_cp = pltpu.CompilerParams()
if "needs_layout_passes" in pltpu.CompilerParams.__dataclass_fields__:
    _cp = dataclasses.replace(_cp, needs_layout_passes=False)

NR = 80
L = 16


def _silu(x):
    return x * jax.nn.sigmoid(x)


def _prep_body(nf_ref, lat_ref, wa_ref, wb_ref, wip_ref, be1_ref,
               p_ref, q_ref, lb_ref):
    nf = nf_ref[...]
    p_ref[...] = jnp.dot(nf, wa_ref[...], preferred_element_type=jnp.float32)
    q_ref[...] = jnp.dot(nf, wb_ref[...], preferred_element_type=jnp.float32)
    x = lat_ref[...]
    lb = be1_ref[...]
    for i in range(3):
        for j in range(3):
            ip = jnp.sum(x[:, 3 * i:3 * i + 3] * x[:, 3 * j:3 * j + 3],
                         axis=1, keepdims=True)
            lb = lb + ip * wip_ref[3 * i + j:3 * i + j + 1, :]
    lb_ref[...] = lb


def _tc_prep(nf, lat9, wa, wb, wip, be1):
    return pl.pallas_call(
        _prep_body,
        out_shape=(
            jax.ShapeDtypeStruct((N, H), jnp.float32),
            jax.ShapeDtypeStruct((N, H), jnp.float32),
            jax.ShapeDtypeStruct((G, H), jnp.float32),
        ),
    )(nf, lat9, wa, wb, wip, be1)


K = EPW // CH


@functools.partial(
    pl.kernel,
    mesh=_mesh,
    out_type=jax.ShapeDtypeStruct((E, H), jnp.float32),
    scratch_types=[
        pltpu.VMEM((CH,), jnp.int32), pltpu.VMEM((CH,), jnp.int32),
        pltpu.VMEM((CH,), jnp.int32), pltpu.VMEM((CH,), jnp.int32),
        pltpu.VMEM((CH,), jnp.int32), pltpu.VMEM((CH,), jnp.int32),
        pltpu.VMEM((CH, H), jnp.float32), pltpu.VMEM((CH, H), jnp.float32),
        pltpu.VMEM((CH, H), jnp.float32), pltpu.VMEM((CH, H), jnp.float32),
        pltpu.VMEM((CH, H), jnp.float32), pltpu.VMEM((CH, H), jnp.float32),
        pltpu.SemaphoreType.DMA, pltpu.SemaphoreType.DMA,
        pltpu.SemaphoreType.DMA, pltpu.SemaphoreType.DMA,
        pltpu.SemaphoreType.DMA, pltpu.SemaphoreType.DMA,
    ],
)
def _sc_gather(p_hbm, q_hbm, lb_hbm, src_hbm, dst_hbm, e2g_hbm, out_hbm,
               si0, si1, di0, di1, gi0, gi1,
               a0, a1, b0_, b1_, c0_, c1_,
               semi0, semi1, semg0, semg1, semo0, semo1):
    cid = lax.axis_index("c")
    sid = lax.axis_index("s")
    wid = sid * NC + cid
    base = wid * EPW
    si = (si0, si1)
    di = (di0, di1)
    gi = (gi0, gi1)
    av = (a0, a1)
    bv = (b0_, b1_)
    cv = (c0_, c1_)
    semi = (semi0, semi1)
    semg = (semg0, semg1)
    semo = (semo0, semo1)

    def issue_idx(c, par):
        o = base + c * CH
        pltpu.async_copy(src_hbm.at[pl.ds(o, CH)], si[par], semi[par])
        pltpu.async_copy(dst_hbm.at[pl.ds(o, CH)], di[par], semi[par])
        pltpu.async_copy(e2g_hbm.at[pl.ds(o, CH)], gi[par], semi[par])

    def wait_idx(c, par):
        o = base + c * CH
        pltpu.make_async_copy(src_hbm.at[pl.ds(o, CH)], si[par], semi[par]).wait()
        pltpu.make_async_copy(dst_hbm.at[pl.ds(o, CH)], di[par], semi[par]).wait()
        pltpu.make_async_copy(e2g_hbm.at[pl.ds(o, CH)], gi[par], semi[par]).wait()

    def issue_gather(par):
        pltpu.async_copy(p_hbm.at[si[par]], av[par], semg[par])
        pltpu.async_copy(q_hbm.at[di[par]], bv[par], semg[par])
        pltpu.async_copy(lb_hbm.at[gi[par]], cv[par], semg[par])

    def wait_gather(par):
        pltpu.make_async_copy(p_hbm.at[si[par]], av[par], semg[par]).wait()
        pltpu.make_async_copy(q_hbm.at[di[par]], bv[par], semg[par]).wait()
        pltpu.make_async_copy(lb_hbm.at[gi[par]], cv[par], semg[par]).wait()

    def do_adds(par):
        @pl.loop(0, CH, step=2)
        def _(r):
            for rr in range(2):
                for c in range(H // L):
                    slc = (pl.ds(r + rr, 1), pl.ds(c * L, L))
                    av[par].at[*slc][...] = (av[par].at[*slc][...]
                                             + bv[par].at[*slc][...]
                                             + cv[par].at[*slc][...])

    def issue_out(c, par):
        o = base + c * CH
        pltpu.async_copy(av[par], out_hbm.at[pl.ds(o, CH)], semo[par])

    def wait_out(c, par):
        o = base + c * CH
        pltpu.make_async_copy(av[par], out_hbm.at[pl.ds(o, CH)], semo[par]).wait()

    issue_idx(0, 0)
    wait_idx(0, 0)
    issue_gather(0)
    issue_idx(1, 1)

    @pl.loop(0, K - 1, step=2)
    def _(i):
        for par in range(2):
            c = i + par

            @pl.when(c >= 1)
            def _():
                wait_out(c - 1, 1 - par)

            wait_idx(c + 1, 1 - par)
            issue_gather(1 - par)
            wait_gather(par)

            @pl.when(c <= K - 3)
            def _():
                issue_idx(c + 2, par)

            do_adds(par)
            issue_out(c, par)

    wait_out(K - 2, 1)
    wait_gather(0)
    do_adds(0)
    issue_out(K - 1, 0)
    wait_out(K - 1, 0)


BE = 2000


def _emlp_body(pre_ref, w2_ref, b2_ref, out_ref):
    e1 = _silu(pre_ref[...])
    e2 = jnp.dot(e1.astype(jnp.bfloat16), w2_ref[...].astype(jnp.bfloat16),
                 preferred_element_type=jnp.float32) + b2_ref[...]
    out_ref[...] = _silu(e2)


def _tc_edge_mlp(pre, w2, b2):
    return pl.pallas_call(
        _emlp_body,
        grid=(E // BE,),
        in_specs=[
            pl.BlockSpec((BE, H), lambda i: (i, 0)),
            pl.BlockSpec((H, H), lambda i: (0, 0)),
            pl.BlockSpec((1, H), lambda i: (0, 0)),
        ],
        out_specs=pl.BlockSpec((BE, H), lambda i: (i, 0)),
        out_shape=jax.ShapeDtypeStruct((E, H), jnp.float32),
    )(pre, w2, b2)


@functools.partial(
    pl.kernel,
    mesh=_mesh,
    compiler_params=_cp,
    out_type=(
        jax.ShapeDtypeStruct((NC, N, H), jnp.float32),
        jax.ShapeDtypeStruct((NC, NR, H), jnp.float32),
    ),
    scratch_types=[
        pltpu.VMEM_SHARED((N, H), jnp.float32),
        pltpu.VMEM_SHARED((NR, H), jnp.float32),
        pltpu.VMEM((CH,), jnp.int32),
        pltpu.VMEM((CH,), jnp.int32),
        pltpu.VMEM((CH, H), jnp.float32),
        pltpu.VMEM((CH, H), jnp.float32),
    ],
)
def _sc_scatter(ef_hbm, src_hbm, zs_hbm, sums_hbm, cnt_hbm,
                shared_s, shared_c, idx_v, cidx_v, rows_v, oh_v):
    cid = lax.axis_index("c")
    sid = lax.axis_index("s")

    zrp = 624
    pltpu.sync_copy(zs_hbm.at[pl.ds(sid * zrp, zrp)],
                    shared_s.at[pl.ds(sid * zrp, zrp)])

    @pl.when(sid == 0)
    def _():
        ztail = NS * zrp
        pltpu.sync_copy(zs_hbm.at[pl.ds(ztail, N - ztail)],
                        shared_s.at[pl.ds(ztail, N - ztail)])
        pltpu.sync_copy(zs_hbm.at[pl.ds(0, NR)], shared_c)

    pltpu.sync_copy(zs_hbm.at[pl.ds(0, CH)], oh_v)
    plsc.subcore_barrier()

    base = cid * (E // NC) + sid * EPW
    ones16 = jnp.full((L,), 1.0, jnp.float32)
    zeros16 = jnp.zeros((L,), jnp.float32)

    @pl.loop(0, EPW, step=CH)
    def _(off):
        b0 = base + off
        pltpu.sync_copy(src_hbm.at[pl.ds(b0, CH)], idx_v)
        pltpu.sync_copy(ef_hbm.at[pl.ds(b0, CH)], rows_v)
        pltpu.sync_copy(rows_v, shared_s.at[idx_v], add=True)
        for g in range(CH // L):
            v = idx_v[pl.ds(g * L, L)]
            row = lax.iota(jnp.int32, L) + (g * L)
            lane = lax.bitwise_and(v, 127)
            plsc.store_scatter(oh_v, [row, lane], ones16)
            cidx_v.at[pl.ds(g * L, L)][...] = lax.shift_right_logical(v, 7)
        pltpu.sync_copy(oh_v, shared_c.at[cidx_v], add=True)
        for g in range(CH // L):
            v = idx_v[pl.ds(g * L, L)]
            row = lax.iota(jnp.int32, L) + (g * L)
            lane = lax.bitwise_and(v, 127)
            plsc.store_scatter(oh_v, [row, lane], zeros16)

    plsc.subcore_barrier()
    rp = 624
    pltpu.sync_copy(shared_s.at[pl.ds(sid * rp, rp)],
                    sums_hbm.at[cid, pl.ds(sid * rp, rp)])

    @pl.when(sid == 0)
    def _():
        tail = NS * rp
        pltpu.sync_copy(shared_s.at[pl.ds(tail, N - tail)],
                        sums_hbm.at[cid, pl.ds(tail, N - tail)])
        pltpu.sync_copy(shared_c, cnt_hbm.at[cid])


def _nmlp_body(nf_ref, s_ref, c_ref, w1a_ref, w1b_ref, b1_ref, w2_ref, b2_ref,
               out_ref):
    nf = nf_ref[...]
    s = s_ref[0] + s_ref[1]
    cpacked = c_ref[0] + c_ref[1]
    b = jnp.broadcast_to(cpacked[:, None, :], (NR, H, H)).reshape(NR * H, H)
    lane = lax.broadcasted_iota(jnp.int32, (NR * H, H), 1)
    rowm = lax.broadcasted_iota(jnp.int32, (NR * H, H), 0) % H
    cnt = jnp.sum(jnp.where(lane == rowm, b, 0.0), axis=1, keepdims=True)[:N]
    agg = s / jnp.maximum(cnt, 1.0)
    n1 = _silu(jnp.dot(nf.astype(jnp.bfloat16),
                       w1a_ref[...].astype(jnp.bfloat16),
                       preferred_element_type=jnp.float32)
               + jnp.dot(agg.astype(jnp.bfloat16),
                         w1b_ref[...].astype(jnp.bfloat16),
                         preferred_element_type=jnp.float32)
               + b1_ref[...])
    n2 = _silu(jnp.dot(n1.astype(jnp.bfloat16), w2_ref[...].astype(jnp.bfloat16),
                       preferred_element_type=jnp.float32)
               + b2_ref[...])
    out_ref[...] = nf + n2


def _tc_node_mlp(nf, sums, cnt, w1a, w1b, b1, w2, b2):
    return pl.pallas_call(
        _nmlp_body,
        out_shape=jax.ShapeDtypeStruct((N, H), jnp.float32),
    )(nf, sums, cnt, w1a, w1b, b1, w2, b2)


def kernel(node_features, frac_coords, lattices_rep, edge_index, edge2graph,
           num_atoms, W_e1, b_e1, W_e2, b_e2, W_n1, b_n1, W_n2, b_n2):
    del frac_coords, num_atoms
    lat9 = lattices_rep.reshape(G, 9)
    wa = W_e1[:H]
    wb = W_e1[H:2 * H]
    wip = W_e1[2 * H:]
    be1 = b_e1.reshape(1, H)
    src = edge_index[0]
    dst = edge_index[1]

    p, q, lb = _tc_prep(node_features, lat9, wa, wb, wip, be1)
    pre = _sc_gather(p, q, lb, src, dst, edge2graph)
    ef = _tc_edge_mlp(pre, W_e2, b_e2.reshape(1, H))

    zs = jnp.zeros((N, H), jnp.float32)
    sums, cnt = _sc_scatter(ef, src, zs)

    return _tc_node_mlp(node_features, sums, cnt,
                        W_n1[:H], W_n1[H:], b_n1.reshape(1, H),
                        W_n2, b_n2.reshape(1, H))


# speedup vs baseline: 6.0313x; 2.2320x over previous
import dataclasses
import functools

import jax
import jax.numpy as jnp
from jax import lax
from jax.experimental import pallas as pl
from jax.experimental.pallas import tpu as pltpu
from jax.experimental.pallas import tpu_sc as plsc

N = 10000
E = 320000
G = 500
H = 128

NC = 2
NS = 16
NW = NC * NS
EPW = E // NW
CH = 80
CNTW = 16

_mesh = plsc.VectorSubcoreMesh(core_axis_name="c", subcore_axis_name="s")

_cp = pltpu.CompilerParams()
if "needs_layout_passes" in pltpu.CompilerParams.__dataclass_fields__:
    _cp = dataclasses.replace(_cp, needs_layout_passes=False)

NR = 80
L = 16
RB = 32


def _silu(x):
    return x * jax.nn.sigmoid(x)


def _prep_body(nf_ref, lat_ref, wa_ref, wb_ref, wip_ref, be1_ref,
               p_ref, q_ref, lb_ref):
    nf = nf_ref[...]
    p_ref[...] = jnp.dot(nf, wa_ref[...], preferred_element_type=jnp.float32)
    q_ref[...] = jnp.dot(nf, wb_ref[...], preferred_element_type=jnp.float32)
    x = lat_ref[...]
    lb = be1_ref[...]
    for i in range(3):
        for j in range(3):
            ip = jnp.sum(x[:, 3 * i:3 * i + 3] * x[:, 3 * j:3 * j + 3],
                         axis=1, keepdims=True)
            lb = lb + ip * wip_ref[3 * i + j:3 * i + j + 1, :]
    lb_ref[...] = jnp.broadcast_to(lb[:, None, :], (G, RB, H)).reshape(G * RB, H)


def _tc_prep(nf, lat9, wa, wb, wip, be1):
    return pl.pallas_call(
        _prep_body,
        out_shape=(
            jax.ShapeDtypeStruct((N, H), jnp.float32),
            jax.ShapeDtypeStruct((N, H), jnp.float32),
            jax.ShapeDtypeStruct((G * RB, H), jnp.float32),
        ),
    )(nf, lat9, wa, wb, wip, be1)


K = EPW // CH


@functools.partial(
    pl.kernel,
    mesh=_mesh,
    out_type=jax.ShapeDtypeStruct((E, H), jnp.float32),
    scratch_types=[
        pltpu.VMEM((CH,), jnp.int32), pltpu.VMEM((CH,), jnp.int32),
        pltpu.VMEM((CH,), jnp.int32), pltpu.VMEM((CH,), jnp.int32),
        pltpu.VMEM((CH,), jnp.int32), pltpu.VMEM((CH,), jnp.int32),
        pltpu.VMEM((CH, H), jnp.float32), pltpu.VMEM((CH, H), jnp.float32),
        pltpu.VMEM((CH, H), jnp.float32), pltpu.VMEM((CH, H), jnp.float32),
        pltpu.VMEM((CH, H), jnp.float32), pltpu.VMEM((CH, H), jnp.float32),
        pltpu.SemaphoreType.DMA, pltpu.SemaphoreType.DMA,
        pltpu.SemaphoreType.DMA, pltpu.SemaphoreType.DMA,
        pltpu.SemaphoreType.DMA, pltpu.SemaphoreType.DMA,
    ],
)
def _sc_gather(p_hbm, q_hbm, lb_hbm, src_hbm, dst_hbm, e2g_hbm, out_hbm,
               si0, si1, di0, di1, gi0, gi1,
               a0, a1, b0_, b1_, c0_, c1_,
               semi0, semi1, semg0, semg1, semo0, semo1):
    cid = lax.axis_index("c")
    sid = lax.axis_index("s")
    wid = sid * NC + cid
    base = wid * EPW
    si = (si0, si1)
    di = (di0, di1)
    gi = (gi0, gi1)
    av = (a0, a1)
    bv = (b0_, b1_)
    cv = (c0_, c1_)
    semi = (semi0, semi1)
    semg = (semg0, semg1)
    semo = (semo0, semo1)

    def issue_idx(c, par):
        o = base + c * CH
        pltpu.async_copy(src_hbm.at[pl.ds(o, CH)], si[par], semi[par])
        pltpu.async_copy(dst_hbm.at[pl.ds(o, CH)], di[par], semi[par])
        pltpu.async_copy(e2g_hbm.at[pl.ds(o, CH)], gi[par], semi[par])

    def wait_idx(c, par):
        o = base + c * CH
        pltpu.make_async_copy(src_hbm.at[pl.ds(o, CH)], si[par], semi[par]).wait()
        pltpu.make_async_copy(dst_hbm.at[pl.ds(o, CH)], di[par], semi[par]).wait()
        pltpu.make_async_copy(e2g_hbm.at[pl.ds(o, CH)], gi[par], semi[par]).wait()

    def spread_gi(par):
        for g in range(CH // L):
            vv = gi[par][pl.ds(g * L, L)]
            gi[par].at[pl.ds(g * L, L)][...] = (
                vv * RB + lax.iota(jnp.int32, L) + ((g % (RB // L)) * L))

    def issue_gather(par):
        pltpu.async_copy(p_hbm.at[si[par]], av[par], semg[par])
        pltpu.async_copy(q_hbm.at[di[par]], bv[par], semg[par])
        pltpu.async_copy(lb_hbm.at[gi[par]], cv[par], semg[par])

    def wait_gather(par):
        pltpu.make_async_copy(p_hbm.at[si[par]], av[par], semg[par]).wait()
        pltpu.make_async_copy(q_hbm.at[di[par]], bv[par], semg[par]).wait()
        pltpu.make_async_copy(lb_hbm.at[gi[par]], cv[par], semg[par]).wait()

    def do_adds(par):
        @pl.loop(0, CH, step=2)
        def _(r):
            for rr in range(2):
                for c in range(H // L):
                    slc = (pl.ds(r + rr, 1), pl.ds(c * L, L))
                    av[par].at[*slc][...] = (av[par].at[*slc][...]
                                             + bv[par].at[*slc][...]
                                             + cv[par].at[*slc][...])

    def issue_out(c, par):
        o = base + c * CH
        pltpu.async_copy(av[par], out_hbm.at[pl.ds(o, CH)], semo[par])

    def wait_out(c, par):
        o = base + c * CH
        pltpu.make_async_copy(av[par], out_hbm.at[pl.ds(o, CH)], semo[par]).wait()

    issue_idx(0, 0)
    wait_idx(0, 0)
    spread_gi(0)
    issue_gather(0)
    issue_idx(1, 1)

    @pl.loop(0, K - 1, step=2)
    def _(i):
        for par in range(2):
            c = i + par

            @pl.when(c >= 1)
            def _():
                wait_out(c - 1, 1 - par)

            wait_idx(c + 1, 1 - par)
            spread_gi(1 - par)
            issue_gather(1 - par)
            wait_gather(par)

            @pl.when(c <= K - 3)
            def _():
                issue_idx(c + 2, par)

            do_adds(par)
            issue_out(c, par)

    wait_out(K - 2, 1)
    wait_gather(0)
    do_adds(0)
    issue_out(K - 1, 0)
    wait_out(K - 1, 0)


BE = 2000


def _emlp_body(pre_ref, w2_ref, b2_ref, out_ref):
    e1 = _silu(pre_ref[...])
    e2 = jnp.dot(e1.astype(jnp.bfloat16), w2_ref[...].astype(jnp.bfloat16),
                 preferred_element_type=jnp.float32) + b2_ref[...]
    out_ref[...] = _silu(e2)


def _tc_edge_mlp(pre, w2, b2):
    return pl.pallas_call(
        _emlp_body,
        grid=(E // BE,),
        in_specs=[
            pl.BlockSpec((BE, H), lambda i: (i, 0)),
            pl.BlockSpec((H, H), lambda i: (0, 0)),
            pl.BlockSpec((1, H), lambda i: (0, 0)),
        ],
        out_specs=pl.BlockSpec((BE, H), lambda i: (i, 0)),
        out_shape=jax.ShapeDtypeStruct((E, H), jnp.float32),
    )(pre, w2, b2)


@functools.partial(
    pl.kernel,
    mesh=_mesh,
    compiler_params=_cp,
    out_type=(
        jax.ShapeDtypeStruct((NC, N, H), jnp.float32),
        jax.ShapeDtypeStruct((NC, NR, H), jnp.float32),
    ),
    scratch_types=[
        pltpu.VMEM_SHARED((N, H), jnp.float32),
        pltpu.VMEM_SHARED((NR, H), jnp.float32),
        pltpu.VMEM((CH,), jnp.int32), pltpu.VMEM((CH,), jnp.int32),
        pltpu.VMEM((CH,), jnp.int32), pltpu.VMEM((CH,), jnp.int32),
        pltpu.VMEM((CH,), jnp.int32), pltpu.VMEM((CH,), jnp.int32),
        pltpu.VMEM((CH, H), jnp.float32), pltpu.VMEM((CH, H), jnp.float32),
        pltpu.VMEM((CH, H), jnp.float32), pltpu.VMEM((CH, H), jnp.float32),
        pltpu.SemaphoreType.DMA, pltpu.SemaphoreType.DMA,
        pltpu.SemaphoreType.DMA, pltpu.SemaphoreType.DMA,
        pltpu.SemaphoreType.DMA, pltpu.SemaphoreType.DMA,
    ],
)
def _sc_scatter(ef_hbm, src_hbm, zs_hbm, sums_hbm, cnt_hbm,
                shared_s, shared_c, ix0, ix1, cx0, cx1, ln0, ln1,
                r0, r1, o0, o1,
                semin0, semin1, sems0, sems1, semc0, semc1):
    cid = lax.axis_index("c")
    sid = lax.axis_index("s")
    idx = (ix0, ix1)
    cidx = (cx0, cx1)
    lanes = (ln0, ln1)
    rows = (r0, r1)
    oh = (o0, o1)
    semin = (semin0, semin1)
    sems = (sems0, sems1)
    semc = (semc0, semc1)

    zrp = 624
    pltpu.sync_copy(zs_hbm.at[pl.ds(sid * zrp, zrp)],
                    shared_s.at[pl.ds(sid * zrp, zrp)])

    @pl.when(sid == 0)
    def _():
        ztail = NS * zrp
        pltpu.sync_copy(zs_hbm.at[pl.ds(ztail, N - ztail)],
                        shared_s.at[pl.ds(ztail, N - ztail)])
        pltpu.sync_copy(zs_hbm.at[pl.ds(0, NR)], shared_c)

    pltpu.sync_copy(zs_hbm.at[pl.ds(0, CH)], o0)
    pltpu.sync_copy(zs_hbm.at[pl.ds(0, CH)], o1)
    plsc.subcore_barrier()

    base = cid * (E // NC) + sid * EPW
    ones16 = jnp.full((L,), 1.0, jnp.float32)
    zeros16 = jnp.zeros((L,), jnp.float32)

    def issue_in(c, par):
        o = base + c * CH
        pltpu.async_copy(src_hbm.at[pl.ds(o, CH)], idx[par], semin[par])
        pltpu.async_copy(ef_hbm.at[pl.ds(o, CH)], rows[par], semin[par])

    def wait_in(c, par):
        o = base + c * CH
        pltpu.make_async_copy(src_hbm.at[pl.ds(o, CH)], idx[par], semin[par]).wait()
        pltpu.make_async_copy(ef_hbm.at[pl.ds(o, CH)], rows[par], semin[par]).wait()

    def chunk_body(c, par, last):
        if not last:
            issue_in(c + 1, 1 - par)
        wait_in(c, par)
        pltpu.sync_copy(rows[par], shared_s.at[idx[par]], add=True)
        for g in range(CH // L):
            v = idx[par][pl.ds(g * L, L)]
            row = lax.iota(jnp.int32, L) + (g * L)
            lane = lax.bitwise_and(v, 127)
            plsc.store_scatter(oh[par], [row, lane], ones16)
            cidx[par].at[pl.ds(g * L, L)][...] = lax.shift_right_logical(v, 7)
        pltpu.sync_copy(oh[par], shared_c.at[cidx[par]], add=True)
        for g in range(CH // L):
            v = idx[par][pl.ds(g * L, L)]
            row = lax.iota(jnp.int32, L) + (g * L)
            lane = lax.bitwise_and(v, 127)
            plsc.store_scatter(oh[par], [row, lane], zeros16)

    issue_in(0, 0)

    @pl.loop(0, K - 1, step=2)
    def _(i):
        for par in range(2):
            chunk_body(i + par, par, False)

    chunk_body(K - 1, 0, True)

    plsc.subcore_barrier()
    rp = 624
    pltpu.sync_copy(shared_s.at[pl.ds(sid * rp, rp)],
                    sums_hbm.at[cid, pl.ds(sid * rp, rp)])

    @pl.when(sid == 0)
    def _():
        tail = NS * rp
        pltpu.sync_copy(shared_s.at[pl.ds(tail, N - tail)],
                        sums_hbm.at[cid, pl.ds(tail, N - tail)])
        pltpu.sync_copy(shared_c, cnt_hbm.at[cid])


def _nmlp_body(nf_ref, s_ref, c_ref, w1a_ref, w1b_ref, b1_ref, w2_ref, b2_ref,
               out_ref):
    nf = nf_ref[...]
    s = s_ref[0] + s_ref[1]
    cpacked = c_ref[0] + c_ref[1]
    b = jnp.broadcast_to(cpacked[:, None, :], (NR, H, H)).reshape(NR * H, H)
    lane = lax.broadcasted_iota(jnp.int32, (NR * H, H), 1)
    rowm = lax.broadcasted_iota(jnp.int32, (NR * H, H), 0) % H
    cnt = jnp.sum(jnp.where(lane == rowm, b, 0.0), axis=1, keepdims=True)[:N]
    agg = s / jnp.maximum(cnt, 1.0)
    n1 = _silu(jnp.dot(nf.astype(jnp.bfloat16),
                       w1a_ref[...].astype(jnp.bfloat16),
                       preferred_element_type=jnp.float32)
               + jnp.dot(agg.astype(jnp.bfloat16),
                         w1b_ref[...].astype(jnp.bfloat16),
                         preferred_element_type=jnp.float32)
               + b1_ref[...])
    n2 = _silu(jnp.dot(n1.astype(jnp.bfloat16), w2_ref[...].astype(jnp.bfloat16),
                       preferred_element_type=jnp.float32)
               + b2_ref[...])
    out_ref[...] = nf + n2


def _tc_node_mlp(nf, sums, cnt, w1a, w1b, b1, w2, b2):
    return pl.pallas_call(
        _nmlp_body,
        out_shape=jax.ShapeDtypeStruct((N, H), jnp.float32),
    )(nf, sums, cnt, w1a, w1b, b1, w2, b2)


def kernel(node_features, frac_coords, lattices_rep, edge_index, edge2graph,
           num_atoms, W_e1, b_e1, W_e2, b_e2, W_n1, b_n1, W_n2, b_n2):
    del frac_coords, num_atoms
    lat9 = lattices_rep.reshape(G, 9)
    wa = W_e1[:H]
    wb = W_e1[H:2 * H]
    wip = W_e1[2 * H:]
    be1 = b_e1.reshape(1, H)
    src = edge_index[0]
    dst = edge_index[1]

    p, q, lb = _tc_prep(node_features, lat9, wa, wb, wip, be1)
    pre = _sc_gather(p, q, lb, src, dst, edge2graph)
    ef = _tc_edge_mlp(pre, W_e2, b_e2.reshape(1, H))

    zs = jnp.zeros((N, H), jnp.float32)
    sums, cnt = _sc_scatter(ef, src, zs)

    return _tc_node_mlp(node_features, sums, cnt,
                        W_n1[:H], W_n1[H:], b_n1.reshape(1, H),
                        W_n2, b_n2.reshape(1, H))
